# Initial kernel scaffold; baseline (speedup 1.0000x reference)
import jax
import jax.numpy as jnp
from jax.experimental import pallas as pl


def kernel(x1, x2, edge_index, edge_attr, W1, att_src1, att_dst1, We1, att_edge1, b1, W2, att_src2, att_dst2, We2, att_edge2, b2, Ws1, bs1, Ws2, bs2, Ws3, bs3):
    raise NotImplementedError("write your pallas kernel here")


# speedup vs baseline: 19.8645x; 19.8645x over previous
import functools

import jax
import jax.numpy as jnp
from jax import lax
from jax.experimental import pallas as pl
from jax.experimental.pallas import tpu as pltpu
from jax.experimental.pallas import tpu_sc as plsc

N = 10000
E = 320000
D = 128
H = 128
S = 512
K = 128
NT = 16
CHUNKS = -(-E // (NT * K))
EPT = CHUNKS * K
EPAD = EPT * NT
RB = 80
DB = 400
BM = 1000
G2N = (2 * N) // BM
F32 = jnp.float32


def _prep_body(ea_ref, we1_ref, ae1_ref, we2_ref, ae2_ref,
               o1_ref, o2_ref, f1_ref, f2_ref):
    c1 = jnp.sum(we1_ref[0, :] * ae1_ref[0, :])
    c2 = jnp.sum(we2_ref[0, :] * ae2_ref[0, :])
    mean_ea = jnp.sum(ea_ref[...]) / E
    o1_ref[...] = ea_ref[...] * c1
    o2_ref[...] = ea_ref[...] * c2
    f1_ref[...] = (mean_ea * c1) * jnp.ones((1, 1), F32)
    f2_ref[...] = (mean_ea * c2) * jnp.ones((1, 1), F32)


def _prep(ea_pad2d, we1, ae1, we2, ae2):
    return pl.pallas_call(
        _prep_body,
        out_shape=[jax.ShapeDtypeStruct((EPAD // 128, 128), F32),
                   jax.ShapeDtypeStruct((EPAD // 128, 128), F32),
                   jax.ShapeDtypeStruct((1, 1), F32),
                   jax.ShapeDtypeStruct((1, 1), F32)],
    )(ea_pad2d, we1, ae1.reshape(1, H), we2, ae2.reshape(1, H))


def _mmaug_body(x_ref, w_ref, as_ref, ad_ref, xp_ref, s_ref, d_ref):
    xp = jnp.dot(x_ref[...], w_ref[...], preferred_element_type=F32)
    xp_ref[...] = xp
    s_ref[...] = jnp.sum(xp * as_ref[...], axis=1, keepdims=True)
    d_ref[...] = jnp.sum(xp * ad_ref[...], axis=1, keepdims=True)


def _mmaug(x, w, a_s, a_d):
    return pl.pallas_call(
        _mmaug_body,
        grid=(G2N,),
        in_specs=[pl.BlockSpec((BM, D), lambda i: (i, 0)),
                  pl.BlockSpec((D, H), lambda i: (0, 0)),
                  pl.BlockSpec((1, H), lambda i: (0, 0)),
                  pl.BlockSpec((1, H), lambda i: (0, 0))],
        out_specs=[pl.BlockSpec((BM, H), lambda i: (i, 0)),
                   pl.BlockSpec((BM, 1), lambda i: (i, 0)),
                   pl.BlockSpec((BM, 1), lambda i: (i, 0))],
        out_shape=[jax.ShapeDtypeStruct((2 * N, H), F32),
                   jax.ShapeDtypeStruct((2 * N, 1), F32),
                   jax.ShapeDtypeStruct((2 * N, 1), F32)],
    )(x, w, a_s.reshape(1, H), a_d.reshape(1, H))


def _combine_body(norm, num_ref, den_ref, xp_ref, as_ref, ad_ref, f_ref,
                  b_ref, o_ref):
    a = as_ref[...] + ad_ref[...] + f_ref[0, 0]
    a = jnp.where(a > 0, a, 0.2 * a)
    exs = jnp.exp(a)
    h = ((num_ref[...] + exs * xp_ref[...])
         / (den_ref[...] + exs + 1e-16) + b_ref[...])
    h = jnp.maximum(h, 0.0)
    if norm:
        nrm = jnp.sqrt(jnp.sum(h * h, axis=1, keepdims=True))
        h = h / jnp.maximum(nrm, 1e-12)
    o_ref[...] = h


def _combine(num, den, xps, asrc, adst, fill, b, norm):
    return pl.pallas_call(
        functools.partial(_combine_body, norm),
        grid=(G2N,),
        in_specs=[pl.BlockSpec((BM, H), lambda i: (i, 0)),
                  pl.BlockSpec((BM, 1), lambda i: (i, 0)),
                  pl.BlockSpec((BM, H), lambda i: (i, 0)),
                  pl.BlockSpec((BM, 1), lambda i: (i, 0)),
                  pl.BlockSpec((BM, 1), lambda i: (i, 0)),
                  pl.BlockSpec((1, 1), lambda i: (0, 0)),
                  pl.BlockSpec((1, H), lambda i: (0, 0))],
        out_specs=pl.BlockSpec((BM, H), lambda i: (i, 0)),
        out_shape=jax.ShapeDtypeStruct((2 * N, H), F32),
    )(num, den, xps, asrc, adst, fill, b.reshape(1, H))


def _mm_body(x_ref, w_ref, b_ref, o_ref):
    o_ref[...] = jnp.maximum(
        jnp.dot(x_ref[...], w_ref[...], preferred_element_type=F32)
        + b_ref[...], 0.0)


def _mm(x, w, b):
    kin = x.shape[1]
    m = w.shape[1]
    return pl.pallas_call(
        _mm_body,
        grid=(G2N,),
        in_specs=[pl.BlockSpec((BM, kin), lambda i: (i, 0)),
                  pl.BlockSpec((kin, m), lambda i: (0, 0)),
                  pl.BlockSpec((1, m), lambda i: (0, 0))],
        out_specs=pl.BlockSpec((BM, m), lambda i: (i, 0)),
        out_shape=jax.ShapeDtypeStruct((2 * N, m), F32),
    )(x, w, b.reshape(1, m))


def _stats_body(p_ref, sum_ref, sq_ref):
    i = pl.program_id(0)

    @pl.when(i % (G2N // 2) == 0)
    def _():
        sum_ref[...] = jnp.zeros_like(sum_ref)
        sq_ref[...] = jnp.zeros_like(sq_ref)

    x = p_ref[...]
    sum_ref[...] += jnp.sum(x, axis=0, keepdims=True)[None]
    sq_ref[...] += jnp.sum(x * x, axis=0, keepdims=True)[None]


def _apply_body(p_ref, sum_ref, sq_ref, z_ref):
    mu = sum_ref[...][0] / N
    var = (sq_ref[...][0] - N * mu * mu) / (N - 1)
    z_ref[...] = (p_ref[...] - mu) / jnp.sqrt(var)


def _standardize(p):
    half = G2N // 2
    sums, sqs = pl.pallas_call(
        _stats_body,
        grid=(G2N,),
        in_specs=[pl.BlockSpec((BM, S), lambda i: (i, 0))],
        out_specs=[pl.BlockSpec((1, 1, S), lambda i: (i // half, 0, 0)),
                   pl.BlockSpec((1, 1, S), lambda i: (i // half, 0, 0))],
        out_shape=[jax.ShapeDtypeStruct((2, 1, S), F32),
                   jax.ShapeDtypeStruct((2, 1, S), F32)],
    )(p)
    return pl.pallas_call(
        _apply_body,
        grid=(G2N,),
        in_specs=[pl.BlockSpec((BM, S), lambda i: (i, 0)),
                  pl.BlockSpec((1, 1, S), lambda i: (i // half, 0, 0)),
                  pl.BlockSpec((1, 1, S), lambda i: (i // half, 0, 0))],
        out_specs=pl.BlockSpec((BM, S), lambda i: (i, 0)),
        out_shape=jax.ShapeDtypeStruct((2 * N, S), F32),
    )(p, sums, sqs)


_sc_mesh = plsc.VectorSubcoreMesh(core_axis_name="c", subcore_axis_name="s")


@functools.partial(
    pl.kernel,
    out_type=[jax.ShapeDtypeStruct((2 * N, H), F32),
              jax.ShapeDtypeStruct((2 * N,), F32)],
    mesh=_sc_mesh,
    compiler_params=pltpu.CompilerParams(needs_layout_passes=False),
    scratch_types=[
        pltpu.VMEM((K,), jnp.int32),
        pltpu.VMEM((K,), jnp.int32),
        pltpu.VMEM((K,), F32),
        pltpu.VMEM((K,), F32),
        pltpu.VMEM((K,), F32),
        pltpu.VMEM((K, H), F32),
        pltpu.VMEM((N,), F32),
        pltpu.VMEM((N,), F32),
        pltpu.VMEM((RB, H), F32),
        pltpu.VMEM((DB,), F32),
        pltpu.VMEM_SHARED((N, H), F32),
        pltpu.VMEM_SHARED((N,), F32),
        pltpu.SemaphoreType.DMA,
    ],
)
def _sc_agg(xps, srcp, dstp, aep, valp, asrc, adst, num_out, den_out,
            src_v, dst_v, ae_v, val_v, ex_v, rows_v, asrc_t, adst_t,
            zb, zbd, acc, den_sh, sem):
    c = lax.axis_index("c")
    s = lax.axis_index("s")
    cN = c * N
    pltpu.sync_copy(asrc.at[pl.ds(cN, N)], asrc_t)
    pltpu.sync_copy(adst.at[pl.ds(cN, N)], adst_t)
    zv = jnp.zeros((16,), F32)
    for r in range(RB):
        for q in range(H // 16):
            zb[r, pl.ds(q * 16, 16)] = zv
    for q in range(DB // 16):
        zbd[pl.ds(q * 16, 16)] = zv
    nrb = N // RB
    for j in range(-(-nrb // NT)):
        cid = s + NT * j

        @pl.when(cid < nrb)
        def _():
            pltpu.sync_copy(zb, acc.at[pl.ds(cid * RB, RB)])
    ndb = N // DB
    for j in range(-(-ndb // NT)):
        cid = s + NT * j

        @pl.when(cid < ndb)
        def _():
            pltpu.sync_copy(zbd, den_sh.at[pl.ds(cid * DB, DB)])
    plsc.subcore_barrier()
    base0 = s * EPT

    def chunk_body(g, carry):
        base = base0 + g * K
        pltpu.sync_copy(srcp.at[pl.ds(base, K)], src_v)
        pltpu.sync_copy(dstp.at[pl.ds(base, K)], dst_v)
        pltpu.sync_copy(aep.at[pl.ds(base, K)], ae_v)
        pltpu.sync_copy(valp.at[pl.ds(base, K)], val_v)
        for j in range(K // 16):
            sl = pl.ds(j * 16, 16)
            si = src_v[sl]
            di = dst_v[sl]
            av = (plsc.load_gather(asrc_t, [si])
                  + plsc.load_gather(adst_t, [di]) + ae_v[sl])
            av = jnp.where(av > 0, av, 0.2 * av)
            ex_v[sl] = jnp.exp(av) * val_v[sl]
            src_v[sl] = si + cN
        pltpu.sync_copy(ex_v, den_sh.at[dst_v], add=True)
        pltpu.async_copy(xps.at[src_v], rows_v, sem).wait()

        def scale_body(i, carry2):
            e = plsc.load_gather(ex_v, [jnp.full((16,), i, jnp.int32)])
            for q in range(H // 16):
                sl = pl.ds(q * 16, 16)
                rows_v[i, sl] = rows_v[i, sl] * e
            return carry2

        lax.fori_loop(0, K, scale_body, 0)
        pltpu.sync_copy(rows_v, acc.at[dst_v], add=True)
        return carry

    lax.fori_loop(0, CHUNKS, chunk_body, 0)
    plsc.subcore_barrier()
    for j in range(-(-nrb // NT)):
        cid = s + NT * j

        @pl.when(cid < nrb)
        def _():
            pltpu.sync_copy(acc.at[pl.ds(cid * RB, RB)],
                            num_out.at[pl.ds(cN + cid * RB, RB)])
    for j in range(-(-ndb // NT)):
        cid = s + NT * j

        @pl.when(cid < ndb)
        def _():
            pltpu.sync_copy(den_sh.at[pl.ds(cid * DB, DB)], zbd)
            pltpu.sync_copy(zbd, den_out.at[pl.ds(cN + cid * DB, DB)])


def kernel(x1, x2, edge_index, edge_attr,
           W1, att_src1, att_dst1, We1, att_edge1, b1,
           W2, att_src2, att_dst2, We2, att_edge2, b2,
           Ws1, bs1, Ws2, bs2, Ws3, bs3):
    src = edge_index[0]
    dst = edge_index[1]
    npad = EPAD - E
    pad_idx = (jnp.arange(npad, dtype=jnp.int32) % N)
    srcp = jnp.concatenate([src, pad_idx])
    dstp = jnp.concatenate([dst, pad_idx])
    valp = jnp.concatenate([jnp.ones((E,), F32), jnp.zeros((npad,), F32)])
    ea_pad = jnp.concatenate([edge_attr[:, 0], jnp.zeros((npad,), F32)])

    ae1p2d, ae2p2d, fill1, fill2 = _prep(
        ea_pad.reshape(EPAD // 128, 128), We1, att_edge1, We2, att_edge2)
    ae1p = ae1p2d.reshape(EPAD)
    ae2p = ae2p2d.reshape(EPAD)

    xs = jnp.concatenate([x1, x2], axis=0)

    xps1, asrc1, adst1 = _mmaug(xs, W1, att_src1, att_dst1)
    num1, den1 = _sc_agg(xps1, srcp, dstp, ae1p, valp,
                         asrc1.reshape(2 * N), adst1.reshape(2 * N))
    h1 = _combine(num1, den1.reshape(2 * N, 1), xps1, asrc1, adst1,
                  fill1, b1, norm=False)

    xps2, asrc2, adst2 = _mmaug(h1, W2, att_src2, att_dst2)
    num2, den2 = _sc_agg(xps2, srcp, dstp, ae2p, valp,
                         asrc2.reshape(2 * N), adst2.reshape(2 * N))
    hn = _combine(num2, den2.reshape(2 * N, 1), xps2, asrc2, adst2,
                  fill2, b2, norm=True)

    p = _mm(hn, Ws1, bs1)
    p = _mm(p, Ws2, bs2)
    p = _mm(p, Ws3, bs3)

    z = _standardize(p)
    return (z[:N][None], z[N:][None])


# speedup vs baseline: 36.7917x; 1.8521x over previous
import functools

import jax
import jax.numpy as jnp
from jax import lax
from jax.experimental import pallas as pl
from jax.experimental.pallas import tpu as pltpu
from jax.experimental.pallas import tpu_sc as plsc

N = 10000
E = 320000
D = 128
H = 128
S = 512
K = 64
NT = 16
CHUNKS = 3 * (-(-E // (3 * NT * K)))
EPT = CHUNKS * K
EPAD = EPT * NT
PKW = 3 * K
RB = 8
DB = 200
BM = 1000
G2N = (2 * N) // BM
F32 = jnp.float32


def _prep_body(ea_ref, we1_ref, ae1_ref, we2_ref, ae2_ref,
               o1_ref, o2_ref, f1_ref, f2_ref):
    c1 = jnp.sum(we1_ref[0, :] * ae1_ref[0, :])
    c2 = jnp.sum(we2_ref[0, :] * ae2_ref[0, :])
    mean_ea = jnp.sum(ea_ref[...]) / E
    o1_ref[...] = ea_ref[...] * c1
    o2_ref[...] = ea_ref[...] * c2
    o1_ref[E // 128:, :] = jnp.full((EPAD // 128 - E // 128, 128), -1e30, F32)
    o2_ref[E // 128:, :] = jnp.full((EPAD // 128 - E // 128, 128), -1e30, F32)
    f1_ref[...] = (mean_ea * c1) * jnp.ones((1, 1), F32)
    f2_ref[...] = (mean_ea * c2) * jnp.ones((1, 1), F32)


def _prep(ea_pad2d, we1, ae1, we2, ae2):
    return pl.pallas_call(
        _prep_body,
        out_shape=[jax.ShapeDtypeStruct((EPAD // 128, 128), F32),
                   jax.ShapeDtypeStruct((EPAD // 128, 128), F32),
                   jax.ShapeDtypeStruct((1, 1), F32),
                   jax.ShapeDtypeStruct((1, 1), F32)],
    )(ea_pad2d, we1, ae1.reshape(1, H), we2, ae2.reshape(1, H))


def _mmaug_body(x_ref, w_ref, as_ref, ad_ref, xp_ref, s_ref, d_ref):
    xp = jnp.dot(x_ref[...], w_ref[...], preferred_element_type=F32)
    xp_ref[...] = xp
    s_ref[...] = jnp.sum(xp * as_ref[...], axis=1, keepdims=True)
    d_ref[...] = jnp.sum(xp * ad_ref[...], axis=1, keepdims=True)


def _mmaug(x, w, a_s, a_d):
    return pl.pallas_call(
        _mmaug_body,
        grid=(G2N,),
        in_specs=[pl.BlockSpec((BM, D), lambda i: (i, 0)),
                  pl.BlockSpec((D, H), lambda i: (0, 0)),
                  pl.BlockSpec((1, H), lambda i: (0, 0)),
                  pl.BlockSpec((1, H), lambda i: (0, 0))],
        out_specs=[pl.BlockSpec((BM, H), lambda i: (i, 0)),
                   pl.BlockSpec((BM, 1), lambda i: (i, 0)),
                   pl.BlockSpec((BM, 1), lambda i: (i, 0))],
        out_shape=[jax.ShapeDtypeStruct((2 * N, H), F32),
                   jax.ShapeDtypeStruct((2 * N, 1), F32),
                   jax.ShapeDtypeStruct((2 * N, 1), F32)],
    )(x, w, a_s.reshape(1, H), a_d.reshape(1, H))


def _combine_body(norm, num_ref, den_ref, xp_ref, as_ref, ad_ref, f_ref,
                  b_ref, o_ref):
    a = as_ref[...] + ad_ref[...] + f_ref[0, 0]
    a = jnp.where(a > 0, a, 0.2 * a)
    exs = jnp.exp(a)
    h = ((num_ref[...] + exs * xp_ref[...])
         / (den_ref[...] + exs + 1e-16) + b_ref[...])
    h = jnp.maximum(h, 0.0)
    if norm:
        nrm = jnp.sqrt(jnp.sum(h * h, axis=1, keepdims=True))
        h = h / jnp.maximum(nrm, 1e-12)
    o_ref[...] = h


def _combine(num, den, xps, asrc, adst, fill, b, norm):
    return pl.pallas_call(
        functools.partial(_combine_body, norm),
        grid=(G2N,),
        in_specs=[pl.BlockSpec((BM, H), lambda i: (i, 0)),
                  pl.BlockSpec((BM, 1), lambda i: (i, 0)),
                  pl.BlockSpec((BM, H), lambda i: (i, 0)),
                  pl.BlockSpec((BM, 1), lambda i: (i, 0)),
                  pl.BlockSpec((BM, 1), lambda i: (i, 0)),
                  pl.BlockSpec((1, 1), lambda i: (0, 0)),
                  pl.BlockSpec((1, H), lambda i: (0, 0))],
        out_specs=pl.BlockSpec((BM, H), lambda i: (i, 0)),
        out_shape=jax.ShapeDtypeStruct((2 * N, H), F32),
    )(num, den, xps, asrc, adst, fill, b.reshape(1, H))


def _mm_body(x_ref, w_ref, b_ref, o_ref):
    o_ref[...] = jnp.maximum(
        jnp.dot(x_ref[...], w_ref[...], preferred_element_type=F32)
        + b_ref[...], 0.0)


def _mm(x, w, b):
    kin = x.shape[1]
    m = w.shape[1]
    return pl.pallas_call(
        _mm_body,
        grid=(G2N,),
        in_specs=[pl.BlockSpec((BM, kin), lambda i: (i, 0)),
                  pl.BlockSpec((kin, m), lambda i: (0, 0)),
                  pl.BlockSpec((1, m), lambda i: (0, 0))],
        out_specs=pl.BlockSpec((BM, m), lambda i: (i, 0)),
        out_shape=jax.ShapeDtypeStruct((2 * N, m), F32),
    )(x, w, b.reshape(1, m))


def _stats_body(p_ref, sum_ref, sq_ref):
    i = pl.program_id(0)

    @pl.when(i % (G2N // 2) == 0)
    def _():
        sum_ref[...] = jnp.zeros_like(sum_ref)
        sq_ref[...] = jnp.zeros_like(sq_ref)

    x = p_ref[...]
    sum_ref[...] += jnp.sum(x, axis=0, keepdims=True)[None]
    sq_ref[...] += jnp.sum(x * x, axis=0, keepdims=True)[None]


def _apply_body(p_ref, sum_ref, sq_ref, z_ref):
    mu = sum_ref[...][0] / N
    var = (sq_ref[...][0] - N * mu * mu) / (N - 1)
    z_ref[...] = (p_ref[...] - mu) / jnp.sqrt(var)


def _standardize(p):
    half = G2N // 2
    sums, sqs = pl.pallas_call(
        _stats_body,
        grid=(G2N,),
        in_specs=[pl.BlockSpec((BM, S), lambda i: (i, 0))],
        out_specs=[pl.BlockSpec((1, 1, S), lambda i: (i // half, 0, 0)),
                   pl.BlockSpec((1, 1, S), lambda i: (i // half, 0, 0))],
        out_shape=[jax.ShapeDtypeStruct((2, 1, S), F32),
                   jax.ShapeDtypeStruct((2, 1, S), F32)],
    )(p)
    return pl.pallas_call(
        _apply_body,
        grid=(G2N,),
        in_specs=[pl.BlockSpec((BM, S), lambda i: (i, 0)),
                  pl.BlockSpec((1, 1, S), lambda i: (i // half, 0, 0)),
                  pl.BlockSpec((1, 1, S), lambda i: (i // half, 0, 0))],
        out_specs=pl.BlockSpec((BM, S), lambda i: (i, 0)),
        out_shape=jax.ShapeDtypeStruct((2 * N, S), F32),
    )(p, sums, sqs)


_sc_mesh = plsc.VectorSubcoreMesh(core_axis_name="c", subcore_axis_name="s")


@functools.partial(
    pl.kernel,
    out_type=[jax.ShapeDtypeStruct((2 * N, H), F32),
              jax.ShapeDtypeStruct((2 * N,), F32)],
    mesh=_sc_mesh,
    compiler_params=pltpu.CompilerParams(needs_layout_passes=False),
    scratch_types=[
        pltpu.VMEM((PKW,), jnp.int32),
        pltpu.VMEM((PKW,), jnp.int32),
        pltpu.VMEM((PKW,), jnp.int32),
        pltpu.VMEM((K,), jnp.int32),
        pltpu.VMEM((K,), jnp.int32),
        pltpu.VMEM((K,), jnp.int32),
        pltpu.VMEM((K,), jnp.int32),
        pltpu.VMEM((K,), jnp.int32),
        pltpu.VMEM((K,), jnp.int32),
        pltpu.VMEM((K,), F32),
        pltpu.VMEM((K,), F32),
        pltpu.VMEM((K,), F32),
        pltpu.VMEM((K, H), F32),
        pltpu.VMEM((K, H), F32),
        pltpu.VMEM((K, H), F32),
        pltpu.VMEM((N,), F32),
        pltpu.VMEM((N,), F32),
        pltpu.VMEM((RB, H), F32),
        pltpu.VMEM((DB,), F32),
        pltpu.VMEM_SHARED((N, H), F32),
        pltpu.VMEM_SHARED((N,), F32),
        pltpu.SemaphoreType.DMA,
        pltpu.SemaphoreType.DMA,
        pltpu.SemaphoreType.DMA,
        pltpu.SemaphoreType.DMA,
    ],
)
def _sc_agg(xps, packed, asrc, adst, num_out, den_out,
            comb0, comb1, comb2, srcg0, srcg1, srcg2,
            dstw0, dstw1, dstw2, exb0, exb1, exb2,
            rows0, rows1, rows2, asrc_t, adst_t, zb, zbd, acc, den_sh,
            sem_i, sem_g, sem_s, sem_d):
    comb = (comb0, comb1, comb2)
    srcg = (srcg0, srcg1, srcg2)
    dstw = (dstw0, dstw1, dstw2)
    exb = (exb0, exb1, exb2)
    rows = (rows0, rows1, rows2)
    c = lax.axis_index("c")
    s = lax.axis_index("s")
    cN = c * N
    pltpu.sync_copy(asrc.at[pl.ds(cN, N)], asrc_t)
    pltpu.sync_copy(adst.at[pl.ds(cN, N)], adst_t)
    zv = jnp.zeros((16,), F32)
    for r in range(RB):
        for q in range(H // 16):
            zb[r, pl.ds(q * 16, 16)] = zv
    for q in range(DB // 16):
        zbd[pl.ds(q * 16, 16)] = zv
    nrb = N // RB
    for j in range(-(-nrb // NT)):
        cid = s + NT * j

        @pl.when(cid < nrb)
        def _():
            pltpu.sync_copy(zb, acc.at[pl.ds(cid * RB, RB)])
    ndb = N // DB
    for j in range(-(-ndb // NT)):
        cid = s + NT * j

        @pl.when(cid < ndb)
        def _():
            pltpu.sync_copy(zbd, den_sh.at[pl.ds(cid * DB, DB)])
    plsc.subcore_barrier()
    cbase = s * CHUNKS

    def fire_idx(g, b):
        pltpu.async_copy(packed.at[pl.ds((cbase + g) * PKW, PKW)],
                         comb[b], sem_i)

    def wait_idx(b):
        pltpu.make_async_copy(packed.at[pl.ds(0, PKW)], comb[b], sem_i).wait()

    def alpha(b):
        for j in range(K // 16):
            sl = pl.ds(j * 16, 16)
            si = comb[b][pl.ds(j * 16, 16)]
            di = comb[b][pl.ds(K + j * 16, 16)]
            ae = plsc.bitcast(comb[b][pl.ds(2 * K + j * 16, 16)], F32)
            av = (plsc.load_gather(asrc_t, [si])
                  + plsc.load_gather(adst_t, [di]) + ae)
            av = jnp.where(av > 0, av, 0.2 * av)
            exb[b][sl] = jnp.exp(av)
            srcg[b][sl] = si + cN
            dstw[b][sl] = di

    def scale(b):
        def body(i, carry2):
            e = plsc.load_gather(exb[b], [jnp.full((16,), i, jnp.int32)])
            for q in range(H // 16):
                sl = pl.ds(q * 16, 16)
                rows[b][i, sl] = rows[b][i, sl] * e
            return carry2

        lax.fori_loop(0, K, body, 0)

    def fire_den(b):
        pltpu.async_copy(exb[b], den_sh.at[dstw[b]], sem_d, add=True)

    def wait_den(b):
        pltpu.make_async_copy(exb[b], den_sh.at[dstw[b]], sem_d).wait()

    def fire_gather(b):
        pltpu.async_copy(xps.at[srcg[b]], rows[b], sem_g)

    def wait_gather(b):
        pltpu.make_async_copy(xps.at[srcg[b]], rows[b], sem_g).wait()

    def fire_scatter(b):
        pltpu.async_copy(rows[b], acc.at[dstw[b]], sem_s, add=True)

    def wait_scatter(b):
        pltpu.make_async_copy(rows[b], acc.at[dstw[b]], sem_s).wait()

    fire_idx(0, 0)
    fire_idx(1, 1)
    fire_idx(2, 2)

    def stage(t, g, b):
        nz = t >= 1
        bp = (b + 2) % 3
        wait_idx(b)

        @pl.when(nz)
        def _():
            wait_den(b)
            wait_scatter(b)

        alpha(b)
        fire_idx(g + 3, b)
        fire_den(b)
        fire_gather(b)
        if b == 0:
            @pl.when(nz)
            def _():
                wait_gather(bp)
                scale(bp)
                fire_scatter(bp)
        else:
            wait_gather(bp)
            scale(bp)
            fire_scatter(bp)

    def pipe_body(t, carry):
        stage(t, 3 * t, 0)
        stage(t, 3 * t + 1, 1)
        stage(t, 3 * t + 2, 2)
        return carry

    lax.fori_loop(0, CHUNKS // 3, pipe_body, 0)
    wait_gather(2)
    scale(2)
    fire_scatter(2)
    wait_scatter(0)
    wait_scatter(1)
    wait_scatter(2)
    wait_den(0)
    wait_den(1)
    wait_den(2)
    wait_idx(0)
    wait_idx(1)
    wait_idx(2)
    plsc.subcore_barrier()
    for j in range(-(-nrb // NT)):
        cid = s + NT * j

        @pl.when(cid < nrb)
        def _():
            pltpu.sync_copy(acc.at[pl.ds(cid * RB, RB)],
                            num_out.at[pl.ds(cN + cid * RB, RB)])
    for j in range(-(-ndb // NT)):
        cid = s + NT * j

        @pl.when(cid < ndb)
        def _():
            pltpu.sync_copy(den_sh.at[pl.ds(cid * DB, DB)], zbd)
            pltpu.sync_copy(zbd, den_out.at[pl.ds(cN + cid * DB, DB)])


def kernel(x1, x2, edge_index, edge_attr,
           W1, att_src1, att_dst1, We1, att_edge1, b1,
           W2, att_src2, att_dst2, We2, att_edge2, b2,
           Ws1, bs1, Ws2, bs2, Ws3, bs3):
    src = edge_index[0]
    dst = edge_index[1]
    npad = EPAD - E
    pad_idx = (jnp.arange(npad, dtype=jnp.int32) % N)
    srcp = jnp.concatenate([src, pad_idx])
    dstp = jnp.concatenate([dst, pad_idx])
    ea_pad = jnp.concatenate([edge_attr[:, 0], jnp.zeros((npad,), F32)])

    ae1p2d, ae2p2d, fill1, fill2 = _prep(
        ea_pad.reshape(EPAD // 128, 128), We1, att_edge1, We2, att_edge2)

    def _pack(ae2d):
        trio = jnp.stack(
            [srcp.reshape(-1, K), dstp.reshape(-1, K),
             jax.lax.bitcast_convert_type(ae2d.reshape(EPAD), jnp.int32)
             .reshape(-1, K)], axis=1).reshape(-1)
        return jnp.concatenate([trio, jnp.zeros((3 * PKW,), jnp.int32)])

    packed1 = _pack(ae1p2d)
    packed2 = _pack(ae2p2d)

    xs = jnp.concatenate([x1, x2], axis=0)

    xps1, asrc1, adst1 = _mmaug(xs, W1, att_src1, att_dst1)
    num1, den1 = _sc_agg(xps1, packed1,
                         asrc1.reshape(2 * N), adst1.reshape(2 * N))
    h1 = _combine(num1, den1.reshape(2 * N, 1), xps1, asrc1, adst1,
                  fill1, b1, norm=False)

    xps2, asrc2, adst2 = _mmaug(h1, W2, att_src2, att_dst2)
    num2, den2 = _sc_agg(xps2, packed2,
                         asrc2.reshape(2 * N), adst2.reshape(2 * N))
    hn = _combine(num2, den2.reshape(2 * N, 1), xps2, asrc2, adst2,
                  fill2, b2, norm=True)

    p = _mm(hn, Ws1, bs1)
    p = _mm(p, Ws2, bs2)
    p = _mm(p, Ws3, bs3)

    z = _standardize(p)
    return (z[:N][None], z[N:][None])


# speedup vs baseline: 39.7262x; 1.0798x over previous
import functools

import jax
import jax.numpy as jnp
from jax import lax
from jax.experimental import pallas as pl
from jax.experimental.pallas import tpu as pltpu
from jax.experimental.pallas import tpu_sc as plsc

N = 10000
E = 320000
D = 128
H = 128
S = 512
K = 64
NT = 16
CHUNKS = 3 * (-(-E // (3 * NT * K)))
EPT = CHUNKS * K
EPAD = EPT * NT
PKW = 3 * K
RB = 8
DB = 200
BM = 1000
G2N = (2 * N) // BM
F32 = jnp.float32


def _prep_body(ea_ref, we1_ref, ae1_ref, we2_ref, ae2_ref,
               o1_ref, o2_ref, f1_ref, f2_ref):
    c1 = jnp.sum(we1_ref[0, :] * ae1_ref[0, :])
    c2 = jnp.sum(we2_ref[0, :] * ae2_ref[0, :])
    mean_ea = jnp.sum(ea_ref[...]) / E
    o1_ref[...] = ea_ref[...] * c1
    o2_ref[...] = ea_ref[...] * c2
    o1_ref[E // 128:, :] = jnp.full((EPAD // 128 - E // 128, 128), -1e30, F32)
    o2_ref[E // 128:, :] = jnp.full((EPAD // 128 - E // 128, 128), -1e30, F32)
    f1_ref[...] = (mean_ea * c1) * jnp.ones((1, 1), F32)
    f2_ref[...] = (mean_ea * c2) * jnp.ones((1, 1), F32)


def _prep(ea_pad2d, we1, ae1, we2, ae2):
    return pl.pallas_call(
        _prep_body,
        out_shape=[jax.ShapeDtypeStruct((EPAD // 128, 128), F32),
                   jax.ShapeDtypeStruct((EPAD // 128, 128), F32),
                   jax.ShapeDtypeStruct((1, 1), F32),
                   jax.ShapeDtypeStruct((1, 1), F32)],
    )(ea_pad2d, we1, ae1.reshape(1, H), we2, ae2.reshape(1, H))


def _mmaug_body(x_ref, w_ref, as_ref, ad_ref, xp_ref, s_ref, d_ref):
    xp = jnp.dot(x_ref[...], w_ref[...], preferred_element_type=F32)
    xp_ref[...] = xp
    s_ref[...] = jnp.sum(xp * as_ref[...], axis=1, keepdims=True)
    d_ref[...] = jnp.sum(xp * ad_ref[...], axis=1, keepdims=True)


def _mmaug(x, w, a_s, a_d):
    return pl.pallas_call(
        _mmaug_body,
        grid=(G2N,),
        in_specs=[pl.BlockSpec((BM, D), lambda i: (i, 0)),
                  pl.BlockSpec((D, H), lambda i: (0, 0)),
                  pl.BlockSpec((1, H), lambda i: (0, 0)),
                  pl.BlockSpec((1, H), lambda i: (0, 0))],
        out_specs=[pl.BlockSpec((BM, H), lambda i: (i, 0)),
                   pl.BlockSpec((BM, 1), lambda i: (i, 0)),
                   pl.BlockSpec((BM, 1), lambda i: (i, 0))],
        out_shape=[jax.ShapeDtypeStruct((2 * N, H), F32),
                   jax.ShapeDtypeStruct((2 * N, 1), F32),
                   jax.ShapeDtypeStruct((2 * N, 1), F32)],
    )(x, w, a_s.reshape(1, H), a_d.reshape(1, H))


def _combine_body(norm, num_ref, den_ref, xp_ref, as_ref, ad_ref, f_ref,
                  b_ref, o_ref):
    a = as_ref[...] + ad_ref[...] + f_ref[0, 0]
    a = jnp.where(a > 0, a, 0.2 * a)
    exs = jnp.exp(a)
    h = ((num_ref[...] + exs * xp_ref[...])
         / (den_ref[...] + exs + 1e-16) + b_ref[...])
    h = jnp.maximum(h, 0.0)
    if norm:
        nrm = jnp.sqrt(jnp.sum(h * h, axis=1, keepdims=True))
        h = h / jnp.maximum(nrm, 1e-12)
    o_ref[...] = h


def _combine(num, den, xps, asrc, adst, fill, b, norm):
    return pl.pallas_call(
        functools.partial(_combine_body, norm),
        grid=(G2N,),
        in_specs=[pl.BlockSpec((BM, H), lambda i: (i, 0)),
                  pl.BlockSpec((BM, 1), lambda i: (i, 0)),
                  pl.BlockSpec((BM, H), lambda i: (i, 0)),
                  pl.BlockSpec((BM, 1), lambda i: (i, 0)),
                  pl.BlockSpec((BM, 1), lambda i: (i, 0)),
                  pl.BlockSpec((1, 1), lambda i: (0, 0)),
                  pl.BlockSpec((1, H), lambda i: (0, 0))],
        out_specs=pl.BlockSpec((BM, H), lambda i: (i, 0)),
        out_shape=jax.ShapeDtypeStruct((2 * N, H), F32),
    )(num, den, xps, asrc, adst, fill, b.reshape(1, H))


def _mm_body(x_ref, w_ref, b_ref, o_ref):
    o_ref[...] = jnp.maximum(
        jnp.dot(x_ref[...], w_ref[...], preferred_element_type=F32)
        + b_ref[...], 0.0)


def _mm(x, w, b):
    kin = x.shape[1]
    m = w.shape[1]
    return pl.pallas_call(
        _mm_body,
        grid=(G2N,),
        in_specs=[pl.BlockSpec((BM, kin), lambda i: (i, 0)),
                  pl.BlockSpec((kin, m), lambda i: (0, 0)),
                  pl.BlockSpec((1, m), lambda i: (0, 0))],
        out_specs=pl.BlockSpec((BM, m), lambda i: (i, 0)),
        out_shape=jax.ShapeDtypeStruct((2 * N, m), F32),
    )(x, w, b.reshape(1, m))


def _stats_body(p_ref, sum_ref, sq_ref):
    i = pl.program_id(0)

    @pl.when(i % (G2N // 2) == 0)
    def _():
        sum_ref[...] = jnp.zeros_like(sum_ref)
        sq_ref[...] = jnp.zeros_like(sq_ref)

    x = p_ref[...]
    sum_ref[...] += jnp.sum(x, axis=0, keepdims=True)[None]
    sq_ref[...] += jnp.sum(x * x, axis=0, keepdims=True)[None]


def _apply_body(p_ref, sum_ref, sq_ref, z_ref):
    mu = sum_ref[...][0] / N
    var = (sq_ref[...][0] - N * mu * mu) / (N - 1)
    z_ref[...] = (p_ref[...] - mu) / jnp.sqrt(var)


def _standardize(p):
    half = G2N // 2
    sums, sqs = pl.pallas_call(
        _stats_body,
        grid=(G2N,),
        in_specs=[pl.BlockSpec((BM, S), lambda i: (i, 0))],
        out_specs=[pl.BlockSpec((1, 1, S), lambda i: (i // half, 0, 0)),
                   pl.BlockSpec((1, 1, S), lambda i: (i // half, 0, 0))],
        out_shape=[jax.ShapeDtypeStruct((2, 1, S), F32),
                   jax.ShapeDtypeStruct((2, 1, S), F32)],
    )(p)
    return pl.pallas_call(
        _apply_body,
        grid=(G2N,),
        in_specs=[pl.BlockSpec((BM, S), lambda i: (i, 0)),
                  pl.BlockSpec((1, 1, S), lambda i: (i // half, 0, 0)),
                  pl.BlockSpec((1, 1, S), lambda i: (i // half, 0, 0))],
        out_specs=pl.BlockSpec((BM, S), lambda i: (i, 0)),
        out_shape=jax.ShapeDtypeStruct((2 * N, S), F32),
    )(p, sums, sqs)


_sc_mesh = plsc.VectorSubcoreMesh(core_axis_name="c", subcore_axis_name="s")


@functools.partial(
    pl.kernel,
    out_type=[jax.ShapeDtypeStruct((2 * N, H), F32),
              jax.ShapeDtypeStruct((2 * N,), F32)],
    mesh=_sc_mesh,
    compiler_params=pltpu.CompilerParams(needs_layout_passes=False),
    scratch_types=[
        pltpu.VMEM((PKW,), jnp.int32),
        pltpu.VMEM((PKW,), jnp.int32),
        pltpu.VMEM((PKW,), jnp.int32),
        pltpu.VMEM((K,), jnp.int32),
        pltpu.VMEM((K,), jnp.int32),
        pltpu.VMEM((K,), jnp.int32),
        pltpu.VMEM((K,), jnp.int32),
        pltpu.VMEM((K,), jnp.int32),
        pltpu.VMEM((K,), jnp.int32),
        pltpu.VMEM((K,), F32),
        pltpu.VMEM((K,), F32),
        pltpu.VMEM((K,), F32),
        pltpu.VMEM((K, H), F32),
        pltpu.VMEM((K, H), F32),
        pltpu.VMEM((K, H), F32),
        pltpu.VMEM((N,), F32),
        pltpu.VMEM((N,), F32),
        pltpu.VMEM((RB, H), F32),
        pltpu.VMEM((DB,), F32),
        pltpu.VMEM_SHARED((N, H), F32),
        pltpu.VMEM_SHARED((N,), F32),
        pltpu.SemaphoreType.DMA,
        pltpu.SemaphoreType.DMA,
        pltpu.SemaphoreType.DMA,
        pltpu.SemaphoreType.DMA,
    ],
)
def _sc_agg(xps, packed, asrc, adst, num_out, den_out,
            comb0, comb1, comb2, srcg0, srcg1, srcg2,
            dstw0, dstw1, dstw2, exb0, exb1, exb2,
            rows0, rows1, rows2, asrc_t, adst_t, zb, zbd, acc, den_sh,
            sem_i, sem_g, sem_s, sem_d):
    comb = (comb0, comb1, comb2)
    srcg = (srcg0, srcg1, srcg2)
    dstw = (dstw0, dstw1, dstw2)
    exb = (exb0, exb1, exb2)
    rows = (rows0, rows1, rows2)
    c = lax.axis_index("c")
    s = lax.axis_index("s")
    cN = c * N
    pltpu.sync_copy(asrc.at[pl.ds(cN, N)], asrc_t)
    pltpu.sync_copy(adst.at[pl.ds(cN, N)], adst_t)
    zv = jnp.zeros((16,), F32)
    for r in range(RB):
        for q in range(H // 16):
            zb[r, pl.ds(q * 16, 16)] = zv
    for q in range(DB // 16):
        zbd[pl.ds(q * 16, 16)] = zv
    nrb = N // RB
    for j in range(-(-nrb // NT)):
        cid = s + NT * j

        @pl.when(cid < nrb)
        def _():
            pltpu.sync_copy(zb, acc.at[pl.ds(cid * RB, RB)])
    ndb = N // DB
    for j in range(-(-ndb // NT)):
        cid = s + NT * j

        @pl.when(cid < ndb)
        def _():
            pltpu.sync_copy(zbd, den_sh.at[pl.ds(cid * DB, DB)])
    plsc.subcore_barrier()
    cbase = s * CHUNKS

    def fire_idx(g, b):
        pltpu.async_copy(packed.at[pl.ds((cbase + g) * PKW, PKW)],
                         comb[b], sem_i)

    def wait_idx(b):
        pltpu.make_async_copy(packed.at[pl.ds(0, PKW)], comb[b], sem_i).wait()

    def alpha(b):
        for j in range(K // 16):
            sl = pl.ds(j * 16, 16)
            si = comb[b][pl.ds(j * 16, 16)]
            di = comb[b][pl.ds(K + j * 16, 16)]
            ae = plsc.bitcast(comb[b][pl.ds(2 * K + j * 16, 16)], F32)
            av = (plsc.load_gather(asrc_t, [si])
                  + plsc.load_gather(adst_t, [di]) + ae)
            av = jnp.where(av > 0, av, 0.2 * av)
            exb[b][sl] = jnp.exp(av)
            srcg[b][sl] = si + cN
            dstw[b][sl] = di

    def scale(b):
        @plsc.parallel_loop(0, K, 1, unroll=4)
        def _(i):
            e = plsc.load_gather(exb[b], [jnp.full((16,), i, jnp.int32)])
            for q in range(H // 16):
                sl = pl.ds(q * 16, 16)
                rows[b][i, sl] = rows[b][i, sl] * e

    def fire_den(b):
        pltpu.async_copy(exb[b], den_sh.at[dstw[b]], sem_d, add=True)

    def wait_den(b):
        pltpu.make_async_copy(exb[b], den_sh.at[dstw[b]], sem_d).wait()

    def fire_gather(b):
        pltpu.async_copy(xps.at[srcg[b]], rows[b], sem_g)

    def wait_gather(b):
        pltpu.make_async_copy(xps.at[srcg[b]], rows[b], sem_g).wait()

    def fire_scatter(b):
        pltpu.async_copy(rows[b], acc.at[dstw[b]], sem_s, add=True)

    def wait_scatter(b):
        pltpu.make_async_copy(rows[b], acc.at[dstw[b]], sem_s).wait()

    fire_idx(0, 0)
    fire_idx(1, 1)
    fire_idx(2, 2)

    def stage(t, g, b):
        nz = t >= 1
        bp = (b + 2) % 3
        wait_idx(b)

        @pl.when(nz)
        def _():
            wait_den(b)
            wait_scatter(b)

        alpha(b)
        fire_idx(g + 3, b)
        fire_den(b)
        fire_gather(b)
        if b == 0:
            @pl.when(nz)
            def _():
                wait_gather(bp)
                scale(bp)
                fire_scatter(bp)
        else:
            wait_gather(bp)
            scale(bp)
            fire_scatter(bp)

    def pipe_body(t, carry):
        stage(t, 3 * t, 0)
        stage(t, 3 * t + 1, 1)
        stage(t, 3 * t + 2, 2)
        return carry

    lax.fori_loop(0, CHUNKS // 3, pipe_body, 0)
    wait_gather(2)
    scale(2)
    fire_scatter(2)
    wait_scatter(0)
    wait_scatter(1)
    wait_scatter(2)
    wait_den(0)
    wait_den(1)
    wait_den(2)
    wait_idx(0)
    wait_idx(1)
    wait_idx(2)
    plsc.subcore_barrier()
    for j in range(-(-nrb // NT)):
        cid = s + NT * j

        @pl.when(cid < nrb)
        def _():
            pltpu.sync_copy(acc.at[pl.ds(cid * RB, RB)],
                            num_out.at[pl.ds(cN + cid * RB, RB)])
    for j in range(-(-ndb // NT)):
        cid = s + NT * j

        @pl.when(cid < ndb)
        def _():
            pltpu.sync_copy(den_sh.at[pl.ds(cid * DB, DB)], zbd)
            pltpu.sync_copy(zbd, den_out.at[pl.ds(cN + cid * DB, DB)])


def kernel(x1, x2, edge_index, edge_attr,
           W1, att_src1, att_dst1, We1, att_edge1, b1,
           W2, att_src2, att_dst2, We2, att_edge2, b2,
           Ws1, bs1, Ws2, bs2, Ws3, bs3):
    src = edge_index[0]
    dst = edge_index[1]
    npad = EPAD - E
    pad_idx = (jnp.arange(npad, dtype=jnp.int32) % N)
    srcp = jnp.concatenate([src, pad_idx])
    dstp = jnp.concatenate([dst, pad_idx])
    ea_pad = jnp.concatenate([edge_attr[:, 0], jnp.zeros((npad,), F32)])

    ae1p2d, ae2p2d, fill1, fill2 = _prep(
        ea_pad.reshape(EPAD // 128, 128), We1, att_edge1, We2, att_edge2)

    def _pack(ae2d):
        trio = jnp.stack(
            [srcp.reshape(-1, K), dstp.reshape(-1, K),
             jax.lax.bitcast_convert_type(ae2d.reshape(EPAD), jnp.int32)
             .reshape(-1, K)], axis=1).reshape(-1)
        return jnp.concatenate([trio, jnp.zeros((3 * PKW,), jnp.int32)])

    packed1 = _pack(ae1p2d)
    packed2 = _pack(ae2p2d)

    xs = jnp.concatenate([x1, x2], axis=0)

    xps1, asrc1, adst1 = _mmaug(xs, W1, att_src1, att_dst1)
    num1, den1 = _sc_agg(xps1, packed1,
                         asrc1.reshape(2 * N), adst1.reshape(2 * N))
    h1 = _combine(num1, den1.reshape(2 * N, 1), xps1, asrc1, adst1,
                  fill1, b1, norm=False)

    xps2, asrc2, adst2 = _mmaug(h1, W2, att_src2, att_dst2)
    num2, den2 = _sc_agg(xps2, packed2,
                         asrc2.reshape(2 * N), adst2.reshape(2 * N))
    hn = _combine(num2, den2.reshape(2 * N, 1), xps2, asrc2, adst2,
                  fill2, b2, norm=True)

    p = _mm(hn, Ws1, bs1)
    p = _mm(p, Ws2, bs2)
    p = _mm(p, Ws3, bs3)

    z = _standardize(p)
    return (z[:N][None], z[N:][None])


# speedup vs baseline: 43.5622x; 1.0966x over previous
import functools

import jax
import jax.numpy as jnp
from jax import lax
from jax.experimental import pallas as pl
from jax.experimental.pallas import tpu as pltpu
from jax.experimental.pallas import tpu_sc as plsc

N = 10000
E = 320000
D = 128
H = 128
S = 512
K = 64
NT = 16
CHUNKS = 3 * (-(-E // (3 * NT * K)))
EPT = CHUNKS * K
EPAD = EPT * NT
PKW = 3 * K
RB = 8
DB = 200
BM = 1000
G2N = (2 * N) // BM
F32 = jnp.float32


def _prep_body(ea_ref, we1_ref, ae1_ref, we2_ref, ae2_ref,
               o1_ref, o2_ref, f1_ref, f2_ref):
    c1 = jnp.sum(we1_ref[0, :] * ae1_ref[0, :])
    c2 = jnp.sum(we2_ref[0, :] * ae2_ref[0, :])
    mean_ea = jnp.sum(ea_ref[...]) / E
    o1_ref[...] = ea_ref[...] * c1
    o2_ref[...] = ea_ref[...] * c2
    o1_ref[E // 128:, :] = jnp.full((EPAD // 128 - E // 128, 128), -1e30, F32)
    o2_ref[E // 128:, :] = jnp.full((EPAD // 128 - E // 128, 128), -1e30, F32)
    f1_ref[...] = (mean_ea * c1) * jnp.ones((1, 1), F32)
    f2_ref[...] = (mean_ea * c2) * jnp.ones((1, 1), F32)


def _prep(ea_pad2d, we1, ae1, we2, ae2):
    return pl.pallas_call(
        _prep_body,
        out_shape=[jax.ShapeDtypeStruct((EPAD // 128, 128), F32),
                   jax.ShapeDtypeStruct((EPAD // 128, 128), F32),
                   jax.ShapeDtypeStruct((1, 1), F32),
                   jax.ShapeDtypeStruct((1, 1), F32)],
    )(ea_pad2d, we1, ae1.reshape(1, H), we2, ae2.reshape(1, H))


def _mmaug_body(x_ref, w_ref, as_ref, ad_ref, xp_ref, s_ref, d_ref):
    xp = jnp.dot(x_ref[...], w_ref[...], preferred_element_type=F32)
    xp_ref[...] = xp
    s_ref[...] = jnp.sum(xp * as_ref[...], axis=1, keepdims=True)
    d_ref[...] = jnp.sum(xp * ad_ref[...], axis=1, keepdims=True)


def _mmaug(x, w, a_s, a_d):
    return pl.pallas_call(
        _mmaug_body,
        grid=(G2N,),
        in_specs=[pl.BlockSpec((BM, D), lambda i: (i, 0)),
                  pl.BlockSpec((D, H), lambda i: (0, 0)),
                  pl.BlockSpec((1, H), lambda i: (0, 0)),
                  pl.BlockSpec((1, H), lambda i: (0, 0))],
        out_specs=[pl.BlockSpec((BM, H), lambda i: (i, 0)),
                   pl.BlockSpec((BM, 1), lambda i: (i, 0)),
                   pl.BlockSpec((BM, 1), lambda i: (i, 0))],
        out_shape=[jax.ShapeDtypeStruct((2 * N, H), F32),
                   jax.ShapeDtypeStruct((2 * N, 1), F32),
                   jax.ShapeDtypeStruct((2 * N, 1), F32)],
    )(x, w, a_s.reshape(1, H), a_d.reshape(1, H))


def _gat_out(num_ref, den_ref, xp_ref, as_ref, ad_ref, f_ref, b_ref, norm):
    a = as_ref[...] + ad_ref[...] + f_ref[0, 0]
    a = jnp.where(a > 0, a, 0.2 * a)
    exs = jnp.exp(a)
    h = ((num_ref[...] + exs * xp_ref[...])
         / (den_ref[...] + exs + 1e-16) + b_ref[...])
    h = jnp.maximum(h, 0.0)
    if norm:
        nrm = jnp.sqrt(jnp.sum(h * h, axis=1, keepdims=True))
        h = h / jnp.maximum(nrm, 1e-12)
    return h


def _cmb_mmaug_body(num_ref, den_ref, xp1_ref, as1_ref, ad1_ref, f_ref,
                    bb_ref, w_ref, as_ref, ad_ref, xp_ref, s_ref, d_ref):
    h1 = _gat_out(num_ref, den_ref, xp1_ref, as1_ref, ad1_ref, f_ref,
                  bb_ref, norm=False)
    xp = jnp.dot(h1, w_ref[...], preferred_element_type=F32)
    xp_ref[...] = xp
    s_ref[...] = jnp.sum(xp * as_ref[...], axis=1, keepdims=True)
    d_ref[...] = jnp.sum(xp * ad_ref[...], axis=1, keepdims=True)


def _cmb_mmaug(num, den, xps1, asrc1, adst1, fill, b, w, a_s, a_d):
    row = pl.BlockSpec((BM, H), lambda i: (i, 0))
    col = pl.BlockSpec((BM, 1), lambda i: (i, 0))
    cst = pl.BlockSpec((1, 1), lambda i: (0, 0))
    vec = pl.BlockSpec((1, H), lambda i: (0, 0))
    return pl.pallas_call(
        _cmb_mmaug_body,
        grid=(G2N,),
        in_specs=[row, col, row, col, col, cst, vec,
                  pl.BlockSpec((H, H), lambda i: (0, 0)), vec, vec],
        out_specs=[row, col, col],
        out_shape=[jax.ShapeDtypeStruct((2 * N, H), F32),
                   jax.ShapeDtypeStruct((2 * N, 1), F32),
                   jax.ShapeDtypeStruct((2 * N, 1), F32)],
    )(num, den, xps1, asrc1, adst1, fill, b.reshape(1, H), w,
      a_s.reshape(1, H), a_d.reshape(1, H))


def _tail_body(num_ref, den_ref, xp_ref, as_ref, ad_ref, f_ref, bb_ref,
               w1_ref, b1_ref, w2_ref, b2_ref, w3_ref, b3_ref,
               p_ref, sum_ref, sq_ref):
    hn = _gat_out(num_ref, den_ref, xp_ref, as_ref, ad_ref, f_ref,
                  bb_ref, norm=True)
    p = jnp.maximum(jnp.dot(hn, w1_ref[...], preferred_element_type=F32)
                    + b1_ref[...], 0.0)
    p = jnp.maximum(jnp.dot(p, w2_ref[...], preferred_element_type=F32)
                    + b2_ref[...], 0.0)
    p = jnp.maximum(jnp.dot(p, w3_ref[...], preferred_element_type=F32)
                    + b3_ref[...], 0.0)
    p_ref[...] = p
    i = pl.program_id(0)

    @pl.when(i % (G2N // 2) == 0)
    def _():
        sum_ref[...] = jnp.zeros_like(sum_ref)
        sq_ref[...] = jnp.zeros_like(sq_ref)

    sum_ref[...] += jnp.sum(p, axis=0, keepdims=True)[None]
    sq_ref[...] += jnp.sum(p * p, axis=0, keepdims=True)[None]


def _tail(num, den, xps, asrc, adst, fill, b, Ws1, bs1, Ws2, bs2, Ws3, bs3):
    half = G2N // 2
    row = pl.BlockSpec((BM, H), lambda i: (i, 0))
    col = pl.BlockSpec((BM, 1), lambda i: (i, 0))
    cst = pl.BlockSpec((1, 1), lambda i: (0, 0))
    vec = pl.BlockSpec((1, H), lambda i: (0, 0))
    svec = pl.BlockSpec((1, S), lambda i: (0, 0))
    stat = pl.BlockSpec((1, 1, S), lambda i: (i // half, 0, 0))
    return pl.pallas_call(
        _tail_body,
        grid=(G2N,),
        in_specs=[row, col, row, col, col, cst, vec,
                  pl.BlockSpec((H, S), lambda i: (0, 0)), svec,
                  pl.BlockSpec((S, S), lambda i: (0, 0)), svec,
                  pl.BlockSpec((S, S), lambda i: (0, 0)), svec],
        out_specs=[pl.BlockSpec((BM, S), lambda i: (i, 0)), stat, stat],
        out_shape=[jax.ShapeDtypeStruct((2 * N, S), F32),
                   jax.ShapeDtypeStruct((2, 1, S), F32),
                   jax.ShapeDtypeStruct((2, 1, S), F32)],
    )(num, den, xps, asrc, adst, fill, b.reshape(1, H), Ws1,
      bs1.reshape(1, S), Ws2, bs2.reshape(1, S), Ws3, bs3.reshape(1, S))


def _apply_body(p_ref, sum_ref, sq_ref, z_ref):
    mu = sum_ref[...][0] / N
    var = (sq_ref[...][0] - N * mu * mu) / (N - 1)
    z_ref[...] = (p_ref[...] - mu) / jnp.sqrt(var)


def _standardize(p, sums, sqs):
    half = G2N // 2
    return pl.pallas_call(
        _apply_body,
        grid=(G2N,),
        in_specs=[pl.BlockSpec((BM, S), lambda i: (i, 0)),
                  pl.BlockSpec((1, 1, S), lambda i: (i // half, 0, 0)),
                  pl.BlockSpec((1, 1, S), lambda i: (i // half, 0, 0))],
        out_specs=pl.BlockSpec((BM, S), lambda i: (i, 0)),
        out_shape=jax.ShapeDtypeStruct((2 * N, S), F32),
    )(p, sums, sqs)


_sc_mesh = plsc.VectorSubcoreMesh(core_axis_name="c", subcore_axis_name="s")


@functools.partial(
    pl.kernel,
    out_type=[jax.ShapeDtypeStruct((2 * N, H), F32),
              jax.ShapeDtypeStruct((2 * N,), F32)],
    mesh=_sc_mesh,
    compiler_params=pltpu.CompilerParams(needs_layout_passes=False),
    scratch_types=[
        pltpu.VMEM((PKW,), jnp.int32),
        pltpu.VMEM((PKW,), jnp.int32),
        pltpu.VMEM((PKW,), jnp.int32),
        pltpu.VMEM((K,), jnp.int32),
        pltpu.VMEM((K,), jnp.int32),
        pltpu.VMEM((K,), jnp.int32),
        pltpu.VMEM((K,), jnp.int32),
        pltpu.VMEM((K,), jnp.int32),
        pltpu.VMEM((K,), jnp.int32),
        pltpu.VMEM((K,), F32),
        pltpu.VMEM((K,), F32),
        pltpu.VMEM((K,), F32),
        pltpu.VMEM((K, H), F32),
        pltpu.VMEM((K, H), F32),
        pltpu.VMEM((K, H), F32),
        pltpu.VMEM((N,), F32),
        pltpu.VMEM((N,), F32),
        pltpu.VMEM((RB, H), F32),
        pltpu.VMEM((DB,), F32),
        pltpu.VMEM_SHARED((N, H), F32),
        pltpu.VMEM_SHARED((N,), F32),
        pltpu.SemaphoreType.DMA,
        pltpu.SemaphoreType.DMA,
        pltpu.SemaphoreType.DMA,
        pltpu.SemaphoreType.DMA,
    ],
)
def _sc_agg(xps, packed, asrc, adst, num_out, den_out,
            comb0, comb1, comb2, srcg0, srcg1, srcg2,
            dstw0, dstw1, dstw2, exb0, exb1, exb2,
            rows0, rows1, rows2, asrc_t, adst_t, zb, zbd, acc, den_sh,
            sem_i, sem_g, sem_s, sem_d):
    comb = (comb0, comb1, comb2)
    srcg = (srcg0, srcg1, srcg2)
    dstw = (dstw0, dstw1, dstw2)
    exb = (exb0, exb1, exb2)
    rows = (rows0, rows1, rows2)
    c = lax.axis_index("c")
    s = lax.axis_index("s")
    cN = c * N
    pltpu.sync_copy(asrc.at[pl.ds(cN, N)], asrc_t)
    pltpu.sync_copy(adst.at[pl.ds(cN, N)], adst_t)
    zv = jnp.zeros((16,), F32)
    for r in range(RB):
        for q in range(H // 16):
            zb[r, pl.ds(q * 16, 16)] = zv
    for q in range(DB // 16):
        zbd[pl.ds(q * 16, 16)] = zv
    nrb = N // RB
    for j in range(-(-nrb // NT)):
        cid = s + NT * j

        @pl.when(cid < nrb)
        def _():
            pltpu.sync_copy(zb, acc.at[pl.ds(cid * RB, RB)])
    ndb = N // DB
    for j in range(-(-ndb // NT)):
        cid = s + NT * j

        @pl.when(cid < ndb)
        def _():
            pltpu.sync_copy(zbd, den_sh.at[pl.ds(cid * DB, DB)])
    plsc.subcore_barrier()
    cbase = s * CHUNKS

    def fire_idx(g, b):
        pltpu.async_copy(packed.at[pl.ds((cbase + g) * PKW, PKW)],
                         comb[b], sem_i)

    def wait_idx(b):
        pltpu.make_async_copy(packed.at[pl.ds(0, PKW)], comb[b], sem_i).wait()

    def alpha(b):
        for j in range(K // 16):
            sl = pl.ds(j * 16, 16)
            si = comb[b][pl.ds(j * 16, 16)]
            di = comb[b][pl.ds(K + j * 16, 16)]
            ae = plsc.bitcast(comb[b][pl.ds(2 * K + j * 16, 16)], F32)
            av = (plsc.load_gather(asrc_t, [si])
                  + plsc.load_gather(adst_t, [di]) + ae)
            av = jnp.where(av > 0, av, 0.2 * av)
            exb[b][sl] = jnp.exp(av)
            srcg[b][sl] = si + cN
            dstw[b][sl] = di

    def scale(b):
        @plsc.parallel_loop(0, K, 1, unroll=4)
        def _(i):
            e = plsc.load_gather(exb[b], [jnp.full((16,), i, jnp.int32)])
            for q in range(H // 16):
                sl = pl.ds(q * 16, 16)
                rows[b][i, sl] = rows[b][i, sl] * e

    def fire_den(b):
        pltpu.async_copy(exb[b], den_sh.at[dstw[b]], sem_d, add=True)

    def wait_den(b):
        pltpu.make_async_copy(exb[b], den_sh.at[dstw[b]], sem_d).wait()

    def fire_gather(b):
        pltpu.async_copy(xps.at[srcg[b]], rows[b], sem_g)

    def wait_gather(b):
        pltpu.make_async_copy(xps.at[srcg[b]], rows[b], sem_g).wait()

    def fire_scatter(b):
        pltpu.async_copy(rows[b], acc.at[dstw[b]], sem_s, add=True)

    def wait_scatter(b):
        pltpu.make_async_copy(rows[b], acc.at[dstw[b]], sem_s).wait()

    fire_idx(0, 0)
    fire_idx(1, 1)
    fire_idx(2, 2)

    def stage(t, g, b):
        nz = t >= 1
        bp = (b + 2) % 3
        wait_idx(b)

        @pl.when(nz)
        def _():
            wait_den(b)
            wait_scatter(b)

        alpha(b)
        fire_idx(g + 3, b)
        fire_den(b)
        fire_gather(b)
        if b == 0:
            @pl.when(nz)
            def _():
                wait_gather(bp)
                scale(bp)
                fire_scatter(bp)
        else:
            wait_gather(bp)
            scale(bp)
            fire_scatter(bp)

    def pipe_body(t, carry):
        stage(t, 3 * t, 0)
        stage(t, 3 * t + 1, 1)
        stage(t, 3 * t + 2, 2)
        return carry

    lax.fori_loop(0, CHUNKS // 3, pipe_body, 0)
    wait_gather(2)
    scale(2)
    fire_scatter(2)
    wait_scatter(0)
    wait_scatter(1)
    wait_scatter(2)
    wait_den(0)
    wait_den(1)
    wait_den(2)
    wait_idx(0)
    wait_idx(1)
    wait_idx(2)
    plsc.subcore_barrier()
    for j in range(-(-nrb // NT)):
        cid = s + NT * j

        @pl.when(cid < nrb)
        def _():
            pltpu.sync_copy(acc.at[pl.ds(cid * RB, RB)],
                            num_out.at[pl.ds(cN + cid * RB, RB)])
    for j in range(-(-ndb // NT)):
        cid = s + NT * j

        @pl.when(cid < ndb)
        def _():
            pltpu.sync_copy(den_sh.at[pl.ds(cid * DB, DB)], zbd)
            pltpu.sync_copy(zbd, den_out.at[pl.ds(cN + cid * DB, DB)])


def kernel(x1, x2, edge_index, edge_attr,
           W1, att_src1, att_dst1, We1, att_edge1, b1,
           W2, att_src2, att_dst2, We2, att_edge2, b2,
           Ws1, bs1, Ws2, bs2, Ws3, bs3):
    src = edge_index[0]
    dst = edge_index[1]
    npad = EPAD - E
    pad_idx = (jnp.arange(npad, dtype=jnp.int32) % N)
    srcp = jnp.concatenate([src, pad_idx])
    dstp = jnp.concatenate([dst, pad_idx])
    ea_pad = jnp.concatenate([edge_attr[:, 0], jnp.zeros((npad,), F32)])

    ae1p2d, ae2p2d, fill1, fill2 = _prep(
        ea_pad.reshape(EPAD // 128, 128), We1, att_edge1, We2, att_edge2)

    def _pack(ae2d):
        trio = jnp.stack(
            [srcp.reshape(-1, K), dstp.reshape(-1, K),
             jax.lax.bitcast_convert_type(ae2d.reshape(EPAD), jnp.int32)
             .reshape(-1, K)], axis=1).reshape(-1)
        return jnp.concatenate([trio, jnp.zeros((3 * PKW,), jnp.int32)])

    packed1 = _pack(ae1p2d)
    packed2 = _pack(ae2p2d)

    xs = jnp.concatenate([x1, x2], axis=0)

    xps1, asrc1, adst1 = _mmaug(xs, W1, att_src1, att_dst1)
    num1, den1 = _sc_agg(xps1, packed1,
                         asrc1.reshape(2 * N), adst1.reshape(2 * N))

    xps2, asrc2, adst2 = _cmb_mmaug(num1, den1.reshape(2 * N, 1), xps1,
                                    asrc1, adst1, fill1, b1, W2,
                                    att_src2, att_dst2)
    num2, den2 = _sc_agg(xps2, packed2,
                         asrc2.reshape(2 * N), adst2.reshape(2 * N))

    p, sums, sqs = _tail(num2, den2.reshape(2 * N, 1), xps2, asrc2, adst2,
                         fill2, b2, Ws1, bs1, Ws2, bs2, Ws3, bs3)

    z = _standardize(p, sums, sqs)
    return (z[:N][None], z[N:][None])


# speedup vs baseline: 43.6628x; 1.0023x over previous
import functools

import jax
import jax.numpy as jnp
from jax import lax
from jax.experimental import pallas as pl
from jax.experimental.pallas import tpu as pltpu
from jax.experimental.pallas import tpu_sc as plsc

N = 10000
E = 320000
D = 128
H = 128
S = 512
K = 64
NT = 16
CHUNKS = 3 * (-(-E // (3 * NT * K)))
EPT = CHUNKS * K
EPAD = EPT * NT
PKW = 3 * K
RB = 8
DB = 200
BM = 1000
G2N = (2 * N) // BM
F32 = jnp.float32


def _prep_body(ea_ref, we1_ref, ae1_ref, we2_ref, ae2_ref,
               o1_ref, o2_ref, f1_ref, f2_ref):
    c1 = jnp.sum(we1_ref[0, :] * ae1_ref[0, :])
    c2 = jnp.sum(we2_ref[0, :] * ae2_ref[0, :])
    mean_ea = jnp.sum(ea_ref[...]) / E
    o1_ref[...] = ea_ref[...] * c1
    o2_ref[...] = ea_ref[...] * c2
    o1_ref[E // 128:, :] = jnp.full((EPAD // 128 - E // 128, 128), -1e30, F32)
    o2_ref[E // 128:, :] = jnp.full((EPAD // 128 - E // 128, 128), -1e30, F32)
    f1_ref[...] = (mean_ea * c1) * jnp.ones((1, 1), F32)
    f2_ref[...] = (mean_ea * c2) * jnp.ones((1, 1), F32)


def _prep(ea_pad2d, we1, ae1, we2, ae2):
    return pl.pallas_call(
        _prep_body,
        out_shape=[jax.ShapeDtypeStruct((EPAD // 128, 128), F32),
                   jax.ShapeDtypeStruct((EPAD // 128, 128), F32),
                   jax.ShapeDtypeStruct((1, 1), F32),
                   jax.ShapeDtypeStruct((1, 1), F32)],
    )(ea_pad2d, we1, ae1.reshape(1, H), we2, ae2.reshape(1, H))


def _mmaug_body(x_ref, w_ref, as_ref, ad_ref, xp_ref, s_ref, d_ref):
    xp = jnp.dot(x_ref[...], w_ref[...], preferred_element_type=F32)
    xp_ref[...] = xp
    s_ref[...] = jnp.sum(xp * as_ref[...], axis=1, keepdims=True)
    d_ref[...] = jnp.sum(xp * ad_ref[...], axis=1, keepdims=True)


def _mmaug(x, w, a_s, a_d):
    return pl.pallas_call(
        _mmaug_body,
        grid=(G2N,),
        in_specs=[pl.BlockSpec((BM, D), lambda i: (i, 0)),
                  pl.BlockSpec((D, H), lambda i: (0, 0)),
                  pl.BlockSpec((1, H), lambda i: (0, 0)),
                  pl.BlockSpec((1, H), lambda i: (0, 0))],
        out_specs=[pl.BlockSpec((BM, H), lambda i: (i, 0)),
                   pl.BlockSpec((BM, 1), lambda i: (i, 0)),
                   pl.BlockSpec((BM, 1), lambda i: (i, 0))],
        out_shape=[jax.ShapeDtypeStruct((2 * N, H), F32),
                   jax.ShapeDtypeStruct((2 * N, 1), F32),
                   jax.ShapeDtypeStruct((2 * N, 1), F32)],
    )(x, w, a_s.reshape(1, H), a_d.reshape(1, H))


def _gat_out(num_ref, den_ref, xp_ref, as_ref, ad_ref, f_ref, b_ref, norm):
    a = as_ref[...] + ad_ref[...] + f_ref[0, 0]
    a = jnp.where(a > 0, a, 0.2 * a)
    exs = jnp.exp(a)
    h = ((num_ref[...] + exs * xp_ref[...])
         / (den_ref[...] + exs + 1e-16) + b_ref[...])
    h = jnp.maximum(h, 0.0)
    if norm:
        nrm = jnp.sqrt(jnp.sum(h * h, axis=1, keepdims=True))
        h = h / jnp.maximum(nrm, 1e-12)
    return h


def _cmb_mmaug_body(num_ref, den_ref, xp1_ref, as1_ref, ad1_ref, f_ref,
                    bb_ref, w_ref, as_ref, ad_ref, xp_ref, s_ref, d_ref):
    h1 = _gat_out(num_ref, den_ref, xp1_ref, as1_ref, ad1_ref, f_ref,
                  bb_ref, norm=False)
    xp = jnp.dot(h1, w_ref[...], preferred_element_type=F32)
    xp_ref[...] = xp
    s_ref[...] = jnp.sum(xp * as_ref[...], axis=1, keepdims=True)
    d_ref[...] = jnp.sum(xp * ad_ref[...], axis=1, keepdims=True)


def _cmb_mmaug(num, den, xps1, asrc1, adst1, fill, b, w, a_s, a_d):
    row = pl.BlockSpec((BM, H), lambda i: (i, 0))
    col = pl.BlockSpec((BM, 1), lambda i: (i, 0))
    cst = pl.BlockSpec((1, 1), lambda i: (0, 0))
    vec = pl.BlockSpec((1, H), lambda i: (0, 0))
    return pl.pallas_call(
        _cmb_mmaug_body,
        grid=(G2N,),
        in_specs=[row, col, row, col, col, cst, vec,
                  pl.BlockSpec((H, H), lambda i: (0, 0)), vec, vec],
        out_specs=[row, col, col],
        out_shape=[jax.ShapeDtypeStruct((2 * N, H), F32),
                   jax.ShapeDtypeStruct((2 * N, 1), F32),
                   jax.ShapeDtypeStruct((2 * N, 1), F32)],
    )(num, den, xps1, asrc1, adst1, fill, b.reshape(1, H), w,
      a_s.reshape(1, H), a_d.reshape(1, H))


def _tail_body(num_ref, den_ref, xp_ref, as_ref, ad_ref, f_ref, bb_ref,
               w1_ref, b1_ref, w2_ref, b2_ref, w3_ref, b3_ref,
               p_ref, sum_ref, sq_ref):
    hn = _gat_out(num_ref, den_ref, xp_ref, as_ref, ad_ref, f_ref,
                  bb_ref, norm=True)
    p = jnp.maximum(jnp.dot(hn, w1_ref[...], preferred_element_type=F32)
                    + b1_ref[...], 0.0)
    p = jnp.maximum(jnp.dot(p, w2_ref[...], preferred_element_type=F32)
                    + b2_ref[...], 0.0)
    p = jnp.maximum(jnp.dot(p, w3_ref[...], preferred_element_type=F32)
                    + b3_ref[...], 0.0)
    p_ref[...] = p
    i = pl.program_id(0)

    @pl.when(i % (G2N // 2) == 0)
    def _():
        sum_ref[...] = jnp.zeros_like(sum_ref)
        sq_ref[...] = jnp.zeros_like(sq_ref)

    sum_ref[...] += jnp.sum(p, axis=0, keepdims=True)[None]
    sq_ref[...] += jnp.sum(p * p, axis=0, keepdims=True)[None]


def _tail(num, den, xps, asrc, adst, fill, b, Ws1, bs1, Ws2, bs2, Ws3, bs3):
    half = G2N // 2
    row = pl.BlockSpec((BM, H), lambda i: (i, 0))
    col = pl.BlockSpec((BM, 1), lambda i: (i, 0))
    cst = pl.BlockSpec((1, 1), lambda i: (0, 0))
    vec = pl.BlockSpec((1, H), lambda i: (0, 0))
    svec = pl.BlockSpec((1, S), lambda i: (0, 0))
    stat = pl.BlockSpec((1, 1, S), lambda i: (i // half, 0, 0))
    return pl.pallas_call(
        _tail_body,
        grid=(G2N,),
        in_specs=[row, col, row, col, col, cst, vec,
                  pl.BlockSpec((H, S), lambda i: (0, 0)), svec,
                  pl.BlockSpec((S, S), lambda i: (0, 0)), svec,
                  pl.BlockSpec((S, S), lambda i: (0, 0)), svec],
        out_specs=[pl.BlockSpec((BM, S), lambda i: (i, 0)), stat, stat],
        out_shape=[jax.ShapeDtypeStruct((2 * N, S), F32),
                   jax.ShapeDtypeStruct((2, 1, S), F32),
                   jax.ShapeDtypeStruct((2, 1, S), F32)],
    )(num, den, xps, asrc, adst, fill, b.reshape(1, H), Ws1,
      bs1.reshape(1, S), Ws2, bs2.reshape(1, S), Ws3, bs3.reshape(1, S))


def _apply_body(p_ref, sum_ref, sq_ref, z_ref):
    mu = sum_ref[...][0] / N
    var = (sq_ref[...][0] - N * mu * mu) / (N - 1)
    z_ref[...] = (p_ref[...] - mu) / jnp.sqrt(var)


def _standardize(p, sums, sqs):
    half = G2N // 2
    return pl.pallas_call(
        _apply_body,
        grid=(G2N,),
        in_specs=[pl.BlockSpec((BM, S), lambda i: (i, 0)),
                  pl.BlockSpec((1, 1, S), lambda i: (i // half, 0, 0)),
                  pl.BlockSpec((1, 1, S), lambda i: (i // half, 0, 0))],
        out_specs=pl.BlockSpec((BM, S), lambda i: (i, 0)),
        out_shape=jax.ShapeDtypeStruct((2 * N, S), F32),
    )(p, sums, sqs)


_sc_mesh = plsc.VectorSubcoreMesh(core_axis_name="c", subcore_axis_name="s")


@functools.partial(
    pl.kernel,
    out_type=[jax.ShapeDtypeStruct((2 * N, H), F32),
              jax.ShapeDtypeStruct((2 * N,), F32)],
    mesh=_sc_mesh,
    compiler_params=pltpu.CompilerParams(needs_layout_passes=False),
    scratch_types=[
        pltpu.VMEM((PKW,), jnp.int32),
        pltpu.VMEM((PKW,), jnp.int32),
        pltpu.VMEM((PKW,), jnp.int32),
        pltpu.VMEM((K,), jnp.int32),
        pltpu.VMEM((K,), jnp.int32),
        pltpu.VMEM((K,), jnp.int32),
        pltpu.VMEM((K,), jnp.int32),
        pltpu.VMEM((K,), jnp.int32),
        pltpu.VMEM((K,), jnp.int32),
        pltpu.VMEM((K,), F32),
        pltpu.VMEM((K,), F32),
        pltpu.VMEM((K,), F32),
        pltpu.VMEM((K, H), F32),
        pltpu.VMEM((K, H), F32),
        pltpu.VMEM((K, H), F32),
        pltpu.VMEM((N,), F32),
        pltpu.VMEM((N,), F32),
        pltpu.VMEM((RB, H), F32),
        pltpu.VMEM((DB,), F32),
        pltpu.VMEM_SHARED((N, H), F32),
        pltpu.VMEM_SHARED((N,), F32),
        pltpu.SemaphoreType.DMA,
        pltpu.SemaphoreType.DMA,
        pltpu.SemaphoreType.DMA,
        pltpu.SemaphoreType.DMA,
    ],
)
def _sc_agg(xps, packed, asrc, adst, num_out, den_out,
            comb0, comb1, comb2, srcg0, srcg1, srcg2,
            dstw0, dstw1, dstw2, exb0, exb1, exb2,
            rows0, rows1, rows2, asrc_t, adst_t, zb, zbd, acc, den_sh,
            sem_i, sem_g, sem_s, sem_d):
    comb = (comb0, comb1, comb2)
    srcg = (srcg0, srcg1, srcg2)
    dstw = (dstw0, dstw1, dstw2)
    exb = (exb0, exb1, exb2)
    rows = (rows0, rows1, rows2)
    c = lax.axis_index("c")
    s = lax.axis_index("s")
    cN = c * N
    pltpu.sync_copy(asrc.at[pl.ds(cN, N)], asrc_t)
    pltpu.sync_copy(adst.at[pl.ds(cN, N)], adst_t)
    zv = jnp.zeros((16,), F32)
    for r in range(RB):
        for q in range(H // 16):
            zb[r, pl.ds(q * 16, 16)] = zv
    for q in range(DB // 16):
        zbd[pl.ds(q * 16, 16)] = zv
    nrb = N // RB
    for j in range(-(-nrb // NT)):
        cid = s + NT * j

        @pl.when(cid < nrb)
        def _():
            pltpu.sync_copy(zb, acc.at[pl.ds(cid * RB, RB)])
    ndb = N // DB
    for j in range(-(-ndb // NT)):
        cid = s + NT * j

        @pl.when(cid < ndb)
        def _():
            pltpu.sync_copy(zbd, den_sh.at[pl.ds(cid * DB, DB)])
    plsc.subcore_barrier()
    cbase = s * CHUNKS

    def fire_idx(g, b):
        pltpu.async_copy(packed.at[pl.ds((cbase + g) * PKW, PKW)],
                         comb[b], sem_i)

    def wait_idx(b):
        pltpu.make_async_copy(packed.at[pl.ds(0, PKW)], comb[b], sem_i).wait()

    def alpha(b):
        for j in range(K // 16):
            sl = pl.ds(j * 16, 16)
            si = comb[b][pl.ds(j * 16, 16)]
            di = comb[b][pl.ds(K + j * 16, 16)]
            ae = plsc.bitcast(comb[b][pl.ds(2 * K + j * 16, 16)], F32)
            av = (plsc.load_gather(asrc_t, [si])
                  + plsc.load_gather(adst_t, [di]) + ae)
            av = jnp.where(av > 0, av, 0.2 * av)
            exb[b][sl] = jnp.exp(av)
            srcg[b][sl] = si + cN
            dstw[b][sl] = di

    def scale(b):
        @plsc.parallel_loop(0, K, 1, unroll=8)
        def _(i):
            e = plsc.load_gather(exb[b], [jnp.full((16,), i, jnp.int32)])
            for q in range(H // 16):
                sl = pl.ds(q * 16, 16)
                rows[b][i, sl] = rows[b][i, sl] * e

    def fire_den(b):
        pltpu.async_copy(exb[b], den_sh.at[dstw[b]], sem_d, add=True)

    def wait_den(b):
        pltpu.make_async_copy(exb[b], den_sh.at[dstw[b]], sem_d).wait()

    def fire_gather(b):
        pltpu.async_copy(xps.at[srcg[b]], rows[b], sem_g)

    def wait_gather(b):
        pltpu.make_async_copy(xps.at[srcg[b]], rows[b], sem_g).wait()

    def fire_scatter(b):
        pltpu.async_copy(rows[b], acc.at[dstw[b]], sem_s, add=True)

    def wait_scatter(b):
        pltpu.make_async_copy(rows[b], acc.at[dstw[b]], sem_s).wait()

    fire_idx(0, 0)
    fire_idx(1, 1)
    fire_idx(2, 2)

    def stage(t, g, b):
        nz = t >= 1
        bp = (b + 2) % 3
        wait_idx(b)

        @pl.when(nz)
        def _():
            wait_den(b)
            wait_scatter(b)

        alpha(b)
        fire_idx(g + 3, b)
        fire_den(b)
        fire_gather(b)
        if b == 0:
            @pl.when(nz)
            def _():
                wait_gather(bp)
                scale(bp)
                fire_scatter(bp)
        else:
            wait_gather(bp)
            scale(bp)
            fire_scatter(bp)

    def pipe_body(t, carry):
        stage(t, 3 * t, 0)
        stage(t, 3 * t + 1, 1)
        stage(t, 3 * t + 2, 2)
        return carry

    lax.fori_loop(0, CHUNKS // 3, pipe_body, 0)
    wait_gather(2)
    scale(2)
    fire_scatter(2)
    wait_scatter(0)
    wait_scatter(1)
    wait_scatter(2)
    wait_den(0)
    wait_den(1)
    wait_den(2)
    wait_idx(0)
    wait_idx(1)
    wait_idx(2)
    plsc.subcore_barrier()
    for j in range(-(-nrb // NT)):
        cid = s + NT * j

        @pl.when(cid < nrb)
        def _():
            pltpu.sync_copy(acc.at[pl.ds(cid * RB, RB)],
                            num_out.at[pl.ds(cN + cid * RB, RB)])
    for j in range(-(-ndb // NT)):
        cid = s + NT * j

        @pl.when(cid < ndb)
        def _():
            pltpu.sync_copy(den_sh.at[pl.ds(cid * DB, DB)], zbd)
            pltpu.sync_copy(zbd, den_out.at[pl.ds(cN + cid * DB, DB)])


def kernel(x1, x2, edge_index, edge_attr,
           W1, att_src1, att_dst1, We1, att_edge1, b1,
           W2, att_src2, att_dst2, We2, att_edge2, b2,
           Ws1, bs1, Ws2, bs2, Ws3, bs3):
    src = edge_index[0]
    dst = edge_index[1]
    npad = EPAD - E
    pad_idx = (jnp.arange(npad, dtype=jnp.int32) % N)
    srcp = jnp.concatenate([src, pad_idx])
    dstp = jnp.concatenate([dst, pad_idx])
    ea_pad = jnp.concatenate([edge_attr[:, 0], jnp.zeros((npad,), F32)])

    ae1p2d, ae2p2d, fill1, fill2 = _prep(
        ea_pad.reshape(EPAD // 128, 128), We1, att_edge1, We2, att_edge2)

    def _pack(ae2d):
        trio = jnp.stack(
            [srcp.reshape(-1, K), dstp.reshape(-1, K),
             jax.lax.bitcast_convert_type(ae2d.reshape(EPAD), jnp.int32)
             .reshape(-1, K)], axis=1).reshape(-1)
        return jnp.concatenate([trio, jnp.zeros((3 * PKW,), jnp.int32)])

    packed1 = _pack(ae1p2d)
    packed2 = _pack(ae2p2d)

    xs = jnp.concatenate([x1, x2], axis=0)

    xps1, asrc1, adst1 = _mmaug(xs, W1, att_src1, att_dst1)
    num1, den1 = _sc_agg(xps1, packed1,
                         asrc1.reshape(2 * N), adst1.reshape(2 * N))

    xps2, asrc2, adst2 = _cmb_mmaug(num1, den1.reshape(2 * N, 1), xps1,
                                    asrc1, adst1, fill1, b1, W2,
                                    att_src2, att_dst2)
    num2, den2 = _sc_agg(xps2, packed2,
                         asrc2.reshape(2 * N), adst2.reshape(2 * N))

    p, sums, sqs = _tail(num2, den2.reshape(2 * N, 1), xps2, asrc2, adst2,
                         fill2, b2, Ws1, bs1, Ws2, bs2, Ws3, bs3)

    z = _standardize(p, sums, sqs)
    return (z[:N][None], z[N:][None])
